# Initial kernel scaffold; baseline (speedup 1.0000x reference)
import jax
import jax.numpy as jnp
from jax.experimental import pallas as pl


def kernel(x, edge_index, edge_attr):
    raise NotImplementedError("write your pallas kernel here")


# speedup vs baseline: 44.0209x; 44.0209x over previous
import functools

import jax
import jax.numpy as jnp
from jax import lax
from jax.experimental import pallas as pl
from jax.experimental.pallas import tpu as pltpu
from jax.experimental.pallas import tpu_sc as plsc

N_NODES = 10000
N_EDGES = 320000
NS = 16
L = 16
CH = 128
N_PAD = 10240
NODES_PT = N_PAD // NS
CPT = -(-N_EDGES // (NS * CH))
EPT = CPT * CH
E_PAD = EPT * NS

_mesh = plsc.VectorSubcoreMesh(
    core_axis_name="c", subcore_axis_name="s", num_cores=1)


@functools.partial(
    pl.kernel,
    out_type=jax.ShapeDtypeStruct((N_PAD,), jnp.float32),
    mesh=_mesh,
    compiler_params=pltpu.CompilerParams(needs_layout_passes=False),
    scratch_types=[
        pltpu.VMEM((N_PAD,), jnp.float32),
        pltpu.VMEM((EPT,), jnp.int32),
        pltpu.VMEM((EPT,), jnp.int32),
        pltpu.VMEM((CPT, CH), jnp.int32),
        pltpu.VMEM((EPT,), jnp.float32),
        pltpu.VMEM((EPT,), jnp.float32),
        pltpu.VMEM((CH,), jnp.float32),
        pltpu.VMEM((NODES_PT,), jnp.float32),
        pltpu.VMEM((NODES_PT,), jnp.float32),
        pltpu.VMEM((NODES_PT,), jnp.float32),
        pltpu.VMEM_SHARED((N_PAD,), jnp.float32),
        pltpu.VMEM_SHARED((N_PAD,), jnp.float32),
    ],
)
def _sc_kernel(xcol_hbm, src_hbm, dst1_hbm, dst2_hbm, ea_hbm, out_hbm,
               xcol_v, src_v, dstc_v, dst2d, ea_v, ld_v, ones_v,
               sv, cv, ov, sums_sh, cnts_sh):
    t = lax.axis_index("s")
    nbase = t * NODES_PT

    pltpu.sync_copy(xcol_hbm, xcol_v)
    pltpu.sync_copy(src_hbm.at[t], src_v)
    pltpu.sync_copy(dst1_hbm.at[t], dstc_v)
    pltpu.sync_copy(dst2_hbm.at[t], dst2d)
    pltpu.sync_copy(ea_hbm.at[t], ea_v)

    def zbody(i, _):
        ov[pl.ds(i * L, L)] = jnp.zeros((L,), jnp.float32)
        return 0
    lax.fori_loop(0, NODES_PT // L, zbody, 0)
    pltpu.sync_copy(ov, sums_sh.at[pl.ds(nbase, NODES_PT)])
    pltpu.sync_copy(ov, cnts_sh.at[pl.ds(nbase, NODES_PT)])
    for i in range(CH // L):
        ones_v[pl.ds(i * L, L)] = jnp.ones((L,), jnp.float32)
    plsc.subcore_barrier()

    def cbody(i, _):
        sl = pl.ds(i * L, L)
        xs = plsc.load_gather(xcol_v, [src_v[sl]])
        xd = plsc.load_gather(xcol_v, [dstc_v[sl]])
        ld_v[sl] = (xd - xs) / ea_v[sl]
        return 0
    lax.fori_loop(0, EPT // L, cbody, 0)

    def sbody(j, _):
        idx = dst2d.at[j]
        pltpu.sync_copy(ld_v.at[pl.ds(j * CH, CH)], sums_sh.at[idx], add=True)
        pltpu.sync_copy(ones_v, cnts_sh.at[idx], add=True)
        return 0
    lax.fori_loop(0, CPT, sbody, 0)
    plsc.subcore_barrier()

    pltpu.sync_copy(sums_sh.at[pl.ds(nbase, NODES_PT)], sv)
    pltpu.sync_copy(cnts_sh.at[pl.ds(nbase, NODES_PT)], cv)

    def obody(i, _):
        sl = pl.ds(i * L, L)
        ov[sl] = sv[sl] / jnp.maximum(cv[sl], 1.0)
        return 0
    lax.fori_loop(0, NODES_PT // L, obody, 0)
    pltpu.sync_copy(ov, out_hbm.at[pl.ds(nbase, NODES_PT)])


@jax.jit
def kernel(x, edge_index, edge_attr):
    xcol = jnp.pad(x[:, 0], (0, N_PAD - N_NODES))
    pad = E_PAD - N_EDGES
    src_p = jnp.pad(edge_index[0], (0, pad)).reshape(NS, EPT)
    dst_p = jnp.pad(edge_index[1], (0, pad), constant_values=N_PAD - 1)
    ea_p = jnp.pad(edge_attr[:, 0], (0, pad),
                   constant_values=1.0).reshape(NS, EPT)
    out = _sc_kernel(xcol, src_p, dst_p.reshape(NS, EPT),
                     dst_p.reshape(NS, CPT, CH), ea_p)
    return out[:N_NODES]


# speedup vs baseline: 57.6684x; 1.3100x over previous
import functools

import jax
import jax.numpy as jnp
from jax import lax
from jax.experimental import pallas as pl
from jax.experimental.pallas import tpu as pltpu
from jax.experimental.pallas import tpu_sc as plsc

N_NODES = 10000
N_EDGES = 320000
NS = 16
L = 16
CH = 128
VPC = CH // L
N_PAD = 10240
NODES_PT = N_PAD // NS
CPT = -(-N_EDGES // (NS * CH))
EPT = CPT * CH
E_PAD = EPT * NS

_mesh = plsc.VectorSubcoreMesh(
    core_axis_name="c", subcore_axis_name="s", num_cores=1)


@functools.partial(
    pl.kernel,
    out_type=jax.ShapeDtypeStruct((N_PAD,), jnp.float32),
    mesh=_mesh,
    compiler_params=pltpu.CompilerParams(needs_layout_passes=False),
    scratch_types=[
        pltpu.VMEM((N_PAD,), jnp.float32),
        pltpu.VMEM((CPT, CH), jnp.int32),
        pltpu.VMEM((CPT, CH), jnp.int32),
        pltpu.VMEM((CPT, CH), jnp.float32),
        pltpu.VMEM((CPT, CH), jnp.float32),
        pltpu.VMEM((CH,), jnp.float32),
        pltpu.VMEM((NODES_PT,), jnp.float32),
        pltpu.VMEM((NODES_PT,), jnp.float32),
        pltpu.VMEM((NODES_PT,), jnp.float32),
        pltpu.VMEM_SHARED((N_PAD,), jnp.float32),
        pltpu.VMEM_SHARED((N_PAD,), jnp.float32),
        pltpu.SemaphoreType.DMA,
        pltpu.SemaphoreType.DMA,
    ],
)
def _sc_kernel(xcol_hbm, src_hbm, dst_hbm, ea_hbm, out_hbm,
               xcol_v, src_v, dst_v, ea_v, ld_v, ones_v,
               sv, cv, ov, sums_sh, cnts_sh, stage_sem, scat_sem):
    t = lax.axis_index("s")
    nbase = t * NODES_PT

    c0 = pltpu.async_copy(xcol_hbm, xcol_v, stage_sem)
    c1 = pltpu.async_copy(src_hbm.at[t], src_v, stage_sem)
    c2 = pltpu.async_copy(dst_hbm.at[t], dst_v, stage_sem)
    c3 = pltpu.async_copy(ea_hbm.at[t], ea_v, stage_sem)

    def zbody(i, _):
        ov[pl.ds(i * L, L)] = jnp.zeros((L,), jnp.float32)
        return 0
    lax.fori_loop(0, NODES_PT // L, zbody, 0)
    for i in range(VPC):
        ones_v[pl.ds(i * L, L)] = jnp.ones((L,), jnp.float32)
    pltpu.sync_copy(ov, sums_sh.at[pl.ds(nbase, NODES_PT)])
    pltpu.sync_copy(ov, cnts_sh.at[pl.ds(nbase, NODES_PT)])
    c0.wait()
    c1.wait()
    c2.wait()
    c3.wait()
    plsc.subcore_barrier()

    @plsc.parallel_loop(0, CPT)
    def _compute(j):
        for k in range(VPC):
            sl = pl.ds(k * L, L)
            xs = plsc.load_gather(xcol_v, [src_v[j, sl]])
            xd = plsc.load_gather(xcol_v, [dst_v[j, sl]])
            ld_v[j, sl] = (xd - xs) / ea_v[j, sl]

    def sbody(j, _):
        idx = dst_v.at[j]
        pltpu.async_copy(ld_v.at[j], sums_sh.at[idx], scat_sem, add=True)
        pltpu.async_copy(ones_v, cnts_sh.at[idx], scat_sem, add=True)

        @pl.when(j > 0)
        def _():
            pidx = dst_v.at[j - 1]
            pltpu.make_async_copy(ld_v.at[j - 1], sums_sh.at[pidx],
                                  scat_sem).wait()
            pltpu.make_async_copy(ones_v, cnts_sh.at[pidx], scat_sem).wait()
        return 0
    lax.fori_loop(0, CPT, sbody, 0)
    lidx = dst_v.at[CPT - 1]
    pltpu.make_async_copy(ld_v.at[CPT - 1], sums_sh.at[lidx], scat_sem).wait()
    pltpu.make_async_copy(ones_v, cnts_sh.at[lidx], scat_sem).wait()
    plsc.subcore_barrier()

    pltpu.sync_copy(sums_sh.at[pl.ds(nbase, NODES_PT)], sv)
    pltpu.sync_copy(cnts_sh.at[pl.ds(nbase, NODES_PT)], cv)

    def obody(i, _):
        sl = pl.ds(i * L, L)
        ov[sl] = sv[sl] / jnp.maximum(cv[sl], 1.0)
        return 0
    lax.fori_loop(0, NODES_PT // L, obody, 0)
    pltpu.sync_copy(ov, out_hbm.at[pl.ds(nbase, NODES_PT)])


@jax.jit
def kernel(x, edge_index, edge_attr):
    xcol = jnp.pad(x[:, 0], (0, N_PAD - N_NODES))
    pad = E_PAD - N_EDGES
    src_p = jnp.pad(edge_index[0], (0, pad)).reshape(NS, CPT, CH)
    dst_p = jnp.pad(edge_index[1], (0, pad),
                    constant_values=N_PAD - 1).reshape(NS, CPT, CH)
    ea_p = jnp.pad(edge_attr[:, 0], (0, pad),
                   constant_values=1.0).reshape(NS, CPT, CH)
    out = _sc_kernel(xcol, src_p, dst_p, ea_p)
    return out[:N_NODES]
